# baseline (speedup 1.0000x reference)
import functools

import jax
import jax.numpy as jnp
from jax import lax
from jax.experimental import pallas as pl
from jax.experimental.pallas import tpu as pltpu
from jax.experimental.pallas import tpu_sc as plsc

_NC = 2
_NS = 16
_L = 16
_CB = 8
_NBUF = 3
_CH = 256


def _tc_scan(B, N, F):
    assert B % _CB == 0 and N % _CH == 0 and F % 128 == 0
    cblk = (F // 128 - 1) * 128
    nchunks = B // _CB
    nc = N // _CH
    nv = _CH // 8

    def body(x_hbm, idx_ref, rm_scr, ri_scr, *scr):
        bufs = scr[:_NBUF]
        sems = scr[_NBUF:]

        def start(c):
            cp = pltpu.make_async_copy(
                x_hbm.at[pl.ds(c * _CB, _CB), :, pl.ds(cblk, 128)],
                bufs[c % _NBUF],
                sems[c % _NBUF],
            )
            cp.start()
            return cp

        pending = [start(c) for c in range(_NBUF - 1)]
        pending.append(None)

        def pairmax(a, b):
            gt = b[0] > a[0]
            return jnp.where(gt, b[0], a[0]), jnp.where(gt, b[1], a[1])

        for c in range(nchunks):
            pending[c % _NBUF].wait()
            buf = bufs[c % _NBUF]
            for i in range(_CB):
                chunks = []
                for cc in range(nc):
                    y = buf[i, pl.ds(cc * _CH, _CH), :].reshape(nv, 8, 128)
                    nodes = []
                    for k in range(nv // 2):
                        g0 = jnp.int32(cc * nv + 2 * k)
                        g1 = jnp.int32(cc * nv + 2 * k + 1)
                        gt = y[2 * k + 1] > y[2 * k]
                        nodes.append((jnp.where(gt, y[2 * k + 1], y[2 * k]),
                                      jnp.where(gt, g1, g0)))
                    while len(nodes) > 1:
                        nodes = [pairmax(nodes[k], nodes[k + 1])
                                 for k in range(0, len(nodes), 2)]
                    chunks.append(nodes[0])
                while len(chunks) > 1:
                    chunks = [pairmax(chunks[k], chunks[k + 1])
                              for k in range(0, len(chunks), 2)]
                rm, ri = chunks[0]
                b = c * _CB + i
                rm_scr[pl.ds(b, 1)] = rm.reshape(1, 8, 128)
                ri_scr[pl.ds(b, 1)] = ri.reshape(1, 8, 128)
            if c + _NBUF - 1 < nchunks:
                pending[(c + _NBUF - 1) % _NBUF] = start(c + _NBUF - 1)

        lane127 = lax.broadcasted_iota(jnp.int32, (8, 8, 128), 2) == 127
        sub3 = lax.broadcasted_iota(jnp.int32, (8, 8, 128), 1)
        bio3 = lax.broadcasted_iota(jnp.int32, (8, 8, 128), 0)
        big3 = jnp.full((8, 8, 128), jnp.int32(1 << 30))
        for t in range(B // 8):
            rm8 = rm_scr[pl.ds(t * 8, 8)]
            ri8 = ri_scr[pl.ds(t * 8, 8)]
            m = jnp.max(rm8, axis=1)
            hit = (rm8 == m[:, None, :]) & lane127
            rowv = (t * 8 + bio3) * N + ri8 * 8 + sub3
            n8 = jnp.min(jnp.where(hit, rowv, big3), axis=1)
            idx_ref[pl.ds(t * 8, 8), :] = n8

    return pl.pallas_call(
        body,
        in_specs=[pl.BlockSpec(memory_space=pl.ANY)],
        out_specs=pl.BlockSpec((B, 128), lambda: (0, 0)),
        out_shape=jax.ShapeDtypeStruct((B, 128), jnp.int32),
        scratch_shapes=(
            [
                pltpu.VMEM((B, 8, 128), jnp.float32),
                pltpu.VMEM((B, 8, 128), jnp.int32),
            ]
            + [pltpu.VMEM((_CB, N, 128), jnp.float32)] * _NBUF
            + [pltpu.SemaphoreType.DMA] * _NBUF
        ),
    )


def _sc_gather(B, N, F):
    bpw = 4
    nw = B // bpw
    assert nw <= _NS
    mesh = plsc.VectorSubcoreMesh(core_axis_name="c", subcore_axis_name="s")

    @functools.partial(
        pl.kernel,
        mesh=mesh,
        out_type=jax.ShapeDtypeStruct((B, F), jnp.float32),
        compiler_params=pltpu.CompilerParams(needs_layout_passes=False),
        scratch_types=[
            pltpu.VMEM((bpw, 128), jnp.int32),
            pltpu.VMEM((bpw,), jnp.int32),
            pltpu.VMEM((bpw, F), jnp.float32),
            pltpu.SemaphoreType.DMA,
        ],
    )
    def sc_kernel(in2d, idx_hbm, out, idxbuf, idx_ref, rows_v, sem):
        cid = lax.axis_index("c")
        sid = lax.axis_index("s")

        @pl.when((cid == 0) & (sid < nw))
        def _():
            b0 = sid * bpw
            pltpu.sync_copy(idx_hbm.at[pl.ds(b0, bpw), :], idxbuf)
            lanes = lax.iota(jnp.int32, _L)
            qv = lanes & (bpw - 1)
            c127 = jnp.full((_L,), 127, jnp.int32)
            rows = plsc.load_gather(idxbuf, [qv, c127])
            plsc.store_scatter(idx_ref, [qv], rows, mask=lanes < bpw)
            pltpu.async_copy(in2d.at[idx_ref], rows_v, sem).wait()
            pltpu.sync_copy(rows_v, out.at[pl.ds(b0, bpw)])

    return sc_kernel


def kernel(inputs):
    B, N, F = inputs.shape
    in2d = inputs.reshape(B * N, F)
    idx = _tc_scan(B, N, F)(inputs)
    return _sc_gather(B, N, F)(in2d, idx)
